# baseline (speedup 1.0000x reference)
import functools

import jax
import jax.numpy as jnp
from jax import lax
from jax.experimental import pallas as pl
from jax.experimental.pallas import tpu as pltpu
from jax.experimental.pallas import tpu_sc as plsc

BN_EPS = 1e-5
NC = 2
NS = 16
CH = 128
LANES = 16


def _sc_aggregate(x, src3, dst3, n_pad, cpw0, cpw1):
    n, d = x.shape
    cpw = max(cpw0, cpw1)
    rows_pt = n_pad // NS

    mesh = plsc.VectorSubcoreMesh(core_axis_name="c", subcore_axis_name="s")

    @functools.partial(
        pl.kernel,
        out_type=jax.ShapeDtypeStruct((NC, n_pad, d), jnp.float32),
        mesh=mesh,
        scratch_types=[
            pltpu.VMEM((cpw, CH), jnp.int32),
            pltpu.VMEM((cpw, CH), jnp.int32),
            pltpu.VMEM((CH, d), jnp.float32),
            pltpu.VMEM_SHARED((n_pad, d), jnp.float32),
        ],
    )
    def agg(x_hbm, src_hbm, dst_hbm, out_hbm, src_v, dst_v, rows_v, acc_sh):
        c = lax.axis_index("c")
        s = lax.axis_index("s")
        wid = c * NS + s
        my_cpw = jnp.where(c == 0, cpw0, cpw1)

        zv = jnp.zeros((LANES,), jnp.float32)

        def zrow(rr, carry):
            for g in range(d // LANES):
                rows_v[rr, pl.ds(g * LANES, LANES)] = zv
            return carry

        lax.fori_loop(0, CH, zrow, 0)
        full, rem = divmod(rows_pt, CH)
        for q in range(full):
            pltpu.sync_copy(rows_v, acc_sh.at[pl.ds(s * rows_pt + q * CH, CH)])
        if rem:
            pltpu.sync_copy(
                rows_v.at[pl.ds(0, rem)],
                acc_sh.at[pl.ds(s * rows_pt + full * CH, rem)])

        pltpu.sync_copy(src_hbm.at[wid], src_v)
        pltpu.sync_copy(dst_hbm.at[wid], dst_v)
        plsc.subcore_barrier()

        def body(j, carry):
            pltpu.sync_copy(x_hbm.at[src_v.at[j]], rows_v)
            pltpu.sync_copy(rows_v, acc_sh.at[dst_v.at[j]], add=True)
            return carry

        lax.fori_loop(0, my_cpw, body, 0)
        plsc.subcore_barrier()

        pltpu.sync_copy(
            acc_sh.at[pl.ds(s * rows_pt, rows_pt)],
            out_hbm.at[c, pl.ds(s * rows_pt, rows_pt)],
        )

    return agg(x, src3, dst3)


def _mlp_bn_kernel(inv_n, x_ref, p_ref, w_ref, b_ref, g_ref, bt_ref, o_ref,
                   z_sc, acc):
    ph = pl.program_id(0)
    i = pl.program_id(1)

    @pl.when(ph == 0)
    def _():
        h = x_ref[...] + p_ref[0] + p_ref[1]
        z = lax.dot_general(
            h, w_ref[...], (((1,), (1,)), ((), ())),
            preferred_element_type=jnp.float32,
        ) + b_ref[...]
        z_sc[i] = z
        ssum = jnp.sum(z, axis=0, keepdims=True)
        qsum = jnp.sum(z * z, axis=0, keepdims=True)

        @pl.when(i == 0)
        def _():
            acc[0:1, :] = ssum
            acc[1:2, :] = qsum

        @pl.when(i != 0)
        def _():
            acc[0:1, :] += ssum
            acc[1:2, :] += qsum

    @pl.when(ph == 1)
    def _():
        mean = acc[0:1, :] * inv_n
        var = acc[1:2, :] * inv_n - mean * mean
        scale = lax.rsqrt(var + BN_EPS) * g_ref[...]
        shift = bt_ref[...] - mean * scale
        o_ref[...] = z_sc[i] * scale + shift


def kernel(x, edge_index, adj_norm_sp, W, b, gamma, beta):
    n, d = x.shape
    e = edge_index.shape[1]
    nw = NC * NS

    src = edge_index[0].astype(jnp.int32)
    dst = edge_index[1].astype(jnp.int32)

    cpw = -(-e // (nw * CH))
    cpw1 = (NC * cpw * 2 // 3) // 8 * 8
    cpw0 = NC * cpw - cpw1
    e_pad = nw * cpw * CH
    if e_pad > e:
        src = jnp.concatenate([src, jnp.zeros((e_pad - e,), jnp.int32)])
        dst = jnp.concatenate([dst, jnp.full((e_pad - e,), n, jnp.int32)])

    def split(a):
        e0 = NS * cpw0 * CH
        a0 = a[:e0].reshape(NS, cpw0, CH)
        a0 = jnp.pad(a0, ((0, 0), (0, cpw1 - cpw0), (0, 0)))
        a1 = a[e0:].reshape(NS, cpw1, CH)
        return jnp.concatenate([a0, a1], axis=0)

    src3 = split(src)
    dst3 = split(dst)

    n_pad = -(-n // (NS * 8)) * (NS * 8)
    if n_pad == n:
        n_pad += NS * 8

    partials = _sc_aggregate(x, src3, dst3, n_pad, cpw0, cpw1)

    nb = 5
    r = n // nb
    out = pl.pallas_call(
        functools.partial(_mlp_bn_kernel, 1.0 / n),
        grid=(2, nb),
        in_specs=[
            pl.BlockSpec((r, d), lambda ph, i: ((1 - ph) * i, 0)),
            pl.BlockSpec((NC, r, d), lambda ph, i: (0, (1 - ph) * i, 0)),
            pl.BlockSpec((d, d), lambda ph, i: (0, 0)),
            pl.BlockSpec((1, d), lambda ph, i: (0, 0)),
            pl.BlockSpec((1, d), lambda ph, i: (0, 0)),
            pl.BlockSpec((1, d), lambda ph, i: (0, 0)),
        ],
        out_specs=pl.BlockSpec((r, d), lambda ph, i: (i, 0)),
        out_shape=jax.ShapeDtypeStruct((n, d), jnp.float32),
        scratch_shapes=[
            pltpu.VMEM((nb, r, d), jnp.float32),
            pltpu.VMEM((2, d), jnp.float32),
        ],
    )(x, partials, W, b.reshape(1, d), gamma.reshape(1, d),
      beta.reshape(1, d))

    return out


# speedup vs baseline: 1.1282x; 1.1282x over previous
import functools

import jax
import jax.numpy as jnp
from jax import lax
from jax.experimental import pallas as pl
from jax.experimental.pallas import tpu as pltpu
from jax.experimental.pallas import tpu_sc as plsc

BN_EPS = 1e-5
NC = 2
NS = 16
CH = 128
LANES = 16


def _sc_aggregate(x, src3, dst3, n_pad, cpw):
    n, d = x.shape
    rows_pt = n_pad // NS

    mesh = plsc.VectorSubcoreMesh(core_axis_name="c", subcore_axis_name="s")

    @functools.partial(
        pl.kernel,
        out_type=jax.ShapeDtypeStruct((NC, n_pad, d), jnp.float32),
        mesh=mesh,
        scratch_types=[
            pltpu.VMEM((cpw, CH), jnp.int32),
            pltpu.VMEM((cpw, CH), jnp.int32),
            pltpu.VMEM((CH, d), jnp.float32),
            pltpu.VMEM_SHARED((n_pad, d), jnp.float32),
        ],
    )
    def agg(x_hbm, src_hbm, dst_hbm, out_hbm, src_v, dst_v, rows_v, acc_sh):
        c = lax.axis_index("c")
        s = lax.axis_index("s")
        wid = c * NS + s

        zv = jnp.zeros((LANES,), jnp.float32)

        def zrow(rr, carry):
            for g in range(d // LANES):
                rows_v[rr, pl.ds(g * LANES, LANES)] = zv
            return carry

        lax.fori_loop(0, CH, zrow, 0)
        full, rem = divmod(rows_pt, CH)
        for q in range(full):
            pltpu.sync_copy(rows_v, acc_sh.at[pl.ds(s * rows_pt + q * CH, CH)])
        if rem:
            pltpu.sync_copy(
                rows_v.at[pl.ds(0, rem)],
                acc_sh.at[pl.ds(s * rows_pt + full * CH, rem)])

        pltpu.sync_copy(src_hbm.at[wid], src_v)
        pltpu.sync_copy(dst_hbm.at[wid], dst_v)
        plsc.subcore_barrier()

        def body(j, carry):
            pltpu.sync_copy(x_hbm.at[src_v.at[j]], rows_v)
            pltpu.sync_copy(rows_v, acc_sh.at[dst_v.at[j]], add=True)
            return carry

        lax.fori_loop(0, cpw, body, 0)
        plsc.subcore_barrier()

        pltpu.sync_copy(
            acc_sh.at[pl.ds(s * rows_pt, rows_pt)],
            out_hbm.at[c, pl.ds(s * rows_pt, rows_pt)],
        )

    return agg(x, src3, dst3)


def _mlp_bn_kernel(inv_n, x_ref, p_ref, w_ref, b_ref, g_ref, bt_ref, o_ref,
                   z_sc, acc):
    ph = pl.program_id(0)
    i = pl.program_id(1)

    @pl.when(ph == 0)
    def _():
        h = x_ref[...] + p_ref[0] + p_ref[1]
        z = lax.dot_general(
            h, w_ref[...], (((1,), (1,)), ((), ())),
            preferred_element_type=jnp.float32,
        ) + b_ref[...]
        z_sc[i] = z
        ssum = jnp.sum(z, axis=0, keepdims=True)
        qsum = jnp.sum(z * z, axis=0, keepdims=True)

        @pl.when(i == 0)
        def _():
            acc[0:1, :] = ssum
            acc[1:2, :] = qsum

        @pl.when(i != 0)
        def _():
            acc[0:1, :] += ssum
            acc[1:2, :] += qsum

    @pl.when(ph == 1)
    def _():
        mean = acc[0:1, :] * inv_n
        var = acc[1:2, :] * inv_n - mean * mean
        scale = lax.rsqrt(var + BN_EPS) * g_ref[...]
        shift = bt_ref[...] - mean * scale
        o_ref[...] = z_sc[i] * scale + shift


def kernel(x, edge_index, adj_norm_sp, W, b, gamma, beta):
    n, d = x.shape
    e = edge_index.shape[1]
    nw = NC * NS

    src = edge_index[0].astype(jnp.int32)
    dst = edge_index[1].astype(jnp.int32)

    cpw = -(-e // (nw * CH))
    e_pad = nw * cpw * CH
    if e_pad > e:
        src = jnp.concatenate([src, jnp.zeros((e_pad - e,), jnp.int32)])
        dst = jnp.concatenate([dst, jnp.full((e_pad - e,), n, jnp.int32)])
    src3 = src.reshape(nw, cpw, CH)
    dst3 = dst.reshape(nw, cpw, CH)

    n_pad = -(-n // (NS * 8)) * (NS * 8)
    if n_pad == n:
        n_pad += NS * 8

    partials = _sc_aggregate(x, src3, dst3, n_pad, cpw)

    nb = 5
    r = n // nb
    out = pl.pallas_call(
        functools.partial(_mlp_bn_kernel, 1.0 / n),
        grid=(2, nb),
        in_specs=[
            pl.BlockSpec((r, d), lambda ph, i: ((1 - ph) * i, 0)),
            pl.BlockSpec((NC, r, d), lambda ph, i: (0, (1 - ph) * i, 0)),
            pl.BlockSpec((d, d), lambda ph, i: (0, 0)),
            pl.BlockSpec((1, d), lambda ph, i: (0, 0)),
            pl.BlockSpec((1, d), lambda ph, i: (0, 0)),
            pl.BlockSpec((1, d), lambda ph, i: (0, 0)),
        ],
        out_specs=pl.BlockSpec((r, d), lambda ph, i: (i, 0)),
        out_shape=jax.ShapeDtypeStruct((n, d), jnp.float32),
        scratch_shapes=[
            pltpu.VMEM((nb, r, d), jnp.float32),
            pltpu.VMEM((2, d), jnp.float32),
        ],
    )(x, partials, W, b.reshape(1, d), gamma.reshape(1, d),
      beta.reshape(1, d))

    return out
